# Initial kernel scaffold; baseline (speedup 1.0000x reference)
import jax
import jax.numpy as jnp
from jax.experimental import pallas as pl


def kernel(x, table0, table1, P):
    raise NotImplementedError("write your pallas kernel here")


# speedup vs baseline: 1.4773x; 1.4773x over previous
import functools

import jax
import jax.numpy as jnp
from jax import lax
from jax.experimental import pallas as pl
from jax.experimental.pallas import tpu as pltpu
from jax.experimental.pallas import tpu_sc as plsc

_VOCAB = 1000000
_D = 64
_COEFF = 0.5


_ROWS_PER_BLOCK = 10000


def _merge_body(t0_ref, t1_ref, p_ref, out_ref):
    acc = jnp.dot(t1_ref[...], p_ref[...], preferred_element_type=jnp.float32)
    out_ref[...] = (1.0 - _COEFF) * t0_ref[...] + acc


def _merge_tables(table0, table1, p_scaled):
    grid = (_VOCAB // _ROWS_PER_BLOCK,)
    return pl.pallas_call(
        _merge_body,
        grid=grid,
        in_specs=[
            pl.BlockSpec((_ROWS_PER_BLOCK, _D), lambda i: (i, 0)),
            pl.BlockSpec((_ROWS_PER_BLOCK, _D), lambda i: (i, 0)),
            pl.BlockSpec((_D, _D), lambda i: (0, 0)),
        ],
        out_specs=pl.BlockSpec((_ROWS_PER_BLOCK, _D), lambda i: (i, 0)),
        out_shape=jax.ShapeDtypeStruct((_VOCAB, _D), jnp.float32),
    )(table0, table1, p_scaled)


_N = 16384 * 50
_NW = 32
_PER_W = _N // _NW
_G = 128
_SUPER = 1024
_NG = _SUPER // _G
_NIT = _PER_W // _SUPER


def _gather_merged(merged, idx2d):
    mesh = plsc.VectorSubcoreMesh(core_axis_name="c", subcore_axis_name="s")
    rows_per_w = _PER_W // _G

    @functools.partial(
        pl.kernel,
        mesh=mesh,
        out_type=jax.ShapeDtypeStruct((_N, _D), jnp.float32),
        scratch_types=[
            pltpu.VMEM((_NG, _G), jnp.int32),
            pltpu.VMEM((_SUPER, _D), jnp.float32),
            pltpu.SemaphoreType.DMA,
        ],
        compiler_params=pltpu.CompilerParams(use_tc_tiling_on_sc=False),
    )
    def _gather_kernel(m_hbm, idx_hbm, out_hbm, idx_v, rows_v, sem):
        wid = lax.axis_index("s") * 2 + lax.axis_index("c")
        base_r = wid * rows_per_w

        def body(i, carry):
            row0 = base_r + i * _NG
            pltpu.sync_copy(idx_hbm.at[pl.ds(row0, _NG)], idx_v)
            copies = []
            for j in range(_NG):
                copies.append(
                    pltpu.async_copy(
                        m_hbm.at[idx_v.at[j]],
                        rows_v.at[pl.ds(j * _G, _G)],
                        sem,
                    )
                )
            for c in copies:
                c.wait()
            pltpu.sync_copy(rows_v, out_hbm.at[pl.ds(row0 * _G, _SUPER)])
            return carry

        lax.fori_loop(0, _NIT, body, 0)

    return _gather_kernel(merged, idx2d)


def kernel(x, table0, table1, P):
    p_scaled = (_COEFF * P).astype(jnp.float32)
    merged = _merge_tables(table0, table1, p_scaled)
    idx2d = x.reshape(_N // _G, _G).astype(jnp.int32)
    out = _gather_merged(merged, idx2d)
    return out.reshape(x.shape[0], x.shape[1], _D)


# speedup vs baseline: 3.0921x; 2.0930x over previous
import functools

import jax
import jax.numpy as jnp
from jax import lax
from jax.experimental import pallas as pl
from jax.experimental.pallas import tpu as pltpu
from jax.experimental.pallas import tpu_sc as plsc

_VOCAB = 1000000
_D = 64
_B = 16384
_L = 50
_COEFF = 0.5
_N = _B * _L


_CB = 8192


def _merge_body(t0_ref, t1_ref, w_ref, out_ref):
    a = jnp.concatenate([t0_ref[...], t1_ref[...]], axis=0)
    out_ref[:, : _D] = lax.dot_general(
        a, w_ref[...], (((0,), (0,)), ((), ())),
        preferred_element_type=jnp.float32,
    )


def _merge_tables(t0t, t1t, w):
    grid = (pl.cdiv(_VOCAB, _CB),)
    return pl.pallas_call(
        _merge_body,
        grid=grid,
        in_specs=[
            pl.BlockSpec((_D, _CB), lambda i: (0, i)),
            pl.BlockSpec((_D, _CB), lambda i: (0, i)),
            pl.BlockSpec((2 * _D, _D), lambda i: (0, 0)),
        ],
        out_specs=pl.BlockSpec((_CB, 2 * _D), lambda i: (i, 0)),
        out_shape=jax.ShapeDtypeStruct((_VOCAB, 2 * _D), jnp.float32),
    )(t0t, t1t, w)


_NW = 32
_BW = _B // _NW
_G = 128
_NG = _BW // _G


def _gather_merged(merged, idx3):
    mesh = plsc.VectorSubcoreMesh(core_axis_name="c", subcore_axis_name="s")

    @functools.partial(
        pl.kernel,
        mesh=mesh,
        out_type=jax.ShapeDtypeStruct((_N, 2 * _D), jnp.float32),
        scratch_types=[
            pltpu.VMEM((_NG, _G), jnp.int32),
            pltpu.VMEM((_BW, 2 * _D), jnp.float32),
            pltpu.SemaphoreType.DMA,
        ],
        compiler_params=pltpu.CompilerParams(use_tc_tiling_on_sc=True),
    )
    def _gather_kernel(m_hbm, idx_hbm, out_hbm, idx_v, rows_v, sem):
        wid = lax.axis_index("s") * 2 + lax.axis_index("c")
        b0 = wid * _BW

        def body(l, carry):
            pltpu.sync_copy(idx_hbm.at[l, pl.ds(wid * _NG, _NG)], idx_v)
            copies = []
            for j in range(_NG):
                copies.append(
                    pltpu.async_copy(
                        m_hbm.at[idx_v.at[j]],
                        rows_v.at[pl.ds(j * _G, _G)],
                        sem,
                    )
                )
            for c in copies:
                c.wait()
            pltpu.sync_copy(rows_v, out_hbm.at[pl.ds(l * _B + b0, _BW)])
            return carry

        lax.fori_loop(0, _L, body, 0)

    return _gather_kernel(merged, idx3)


_FB = 2048


def _format_body(g_ref, out_ref):
    out_ref[0] = g_ref[0, :, : _D].T


def _format_out(g3):
    grid = (_L, _B // _FB)
    return pl.pallas_call(
        _format_body,
        grid=grid,
        in_specs=[pl.BlockSpec((1, _FB, 2 * _D), lambda l, i: (l, i, 0))],
        out_specs=pl.BlockSpec((1, _D, _FB), lambda l, i: (l, 0, i)),
        out_shape=jax.ShapeDtypeStruct((_L, _D, _B), jnp.float32),
    )(g3)


def kernel(x, table0, table1, P):
    w = jnp.concatenate(
        [(1.0 - _COEFF) * jnp.eye(_D, dtype=jnp.float32), _COEFF * P], axis=0
    )
    merged = _merge_tables(table0.T, table1.T, w)
    idx3 = x.T.reshape(_L, _B // _G, _G).astype(jnp.int32)
    g = _gather_merged(merged, idx3)
    out = _format_out(g.reshape(_L, _B, 2 * _D))
    return jnp.transpose(out, (2, 0, 1))
